# Initial kernel scaffold; baseline (speedup 1.0000x reference)
import jax
import jax.numpy as jnp
from jax.experimental import pallas as pl


def kernel(x, edge_index, edge_attr, W1, W2, attn, bias):
    raise NotImplementedError("write your pallas kernel here")


# speedup vs baseline: 8.3040x; 8.3040x over previous
import functools

import jax
import jax.numpy as jnp
from jax import lax
from jax.experimental import pallas as pl
from jax.experimental.pallas import tpu as pltpu
from jax.experimental.pallas import tpu_sc as plsc

N = 10000
E = 320000
D = 128
DE = 16
DP = D + 16

NC = 2
NS = 16
NW = NC * NS
EPW = E // NW
CH = 80
NCH = EPW // CH
NP = 10240
ZR = 128

@functools.cache
def _mesh():
    return plsc.VectorSubcoreMesh(
        core_axis_name="c", subcore_axis_name="s", num_cores=NC, num_subcores=NS
    )


BN = 1000


def _prep_body(x_ref, w1x_ref, w2_ref, g_ref, xt_ref):
    xb = x_ref[...]
    g_ref[...] = lax.dot_general(
        xb, w1x_ref[...], (((1,), (1,)), ((), ())),
        preferred_element_type=jnp.float32)
    xt = lax.dot_general(
        xb, w2_ref[...], (((1,), (1,)), ((), ())),
        preferred_element_type=jnp.float32)
    pad = jnp.where(
        lax.broadcasted_iota(jnp.int32, (xb.shape[0], DP - D), 1) == 0,
        1.0, 0.0).astype(jnp.float32)
    xt_ref[...] = jnp.concatenate([xt, pad], axis=1)


def _prep(x, w1x, w2):
    return pl.pallas_call(
        _prep_body,
        grid=(N // BN,),
        in_specs=[
            pl.BlockSpec((BN, D), lambda i: (i, 0)),
            pl.BlockSpec((D, D), lambda i: (0, 0)),
            pl.BlockSpec((D, D), lambda i: (0, 0)),
        ],
        out_specs=[
            pl.BlockSpec((BN, D), lambda i: (i, 0)),
            pl.BlockSpec((BN, DP), lambda i: (i, 0)),
        ],
        out_shape=[
            jax.ShapeDtypeStruct((N, D), jnp.float32),
            jax.ShapeDtypeStruct((N, DP), jnp.float32),
        ],
    )(x, w1x, w2)


@functools.cache
def _gather_rows_kernel():
    return pl.kernel(
        _gather_rows_body,
        out_type=jax.ShapeDtypeStruct((E, D), jnp.float32),
        mesh=_mesh(),
        scratch_types=[
            pltpu.VMEM((CH,), jnp.int32),
            pltpu.VMEM((CH, D), jnp.float32),
            pltpu.SemaphoreType.DMA,
        ],
        compiler_params=pltpu.CompilerParams(use_tc_tiling_on_sc=False),
    )


def _gather_rows_body(g_hbm, src_hbm, out_hbm, idx_v, rows_v, sem):
    wid = lax.axis_index("s") * NC + lax.axis_index("c")
    base0 = wid * EPW

    def body(i, _):
        base = base0 + i * CH
        pltpu.sync_copy(src_hbm.at[pl.ds(base, CH)], idx_v)
        pltpu.async_copy(g_hbm.at[idx_v], rows_v, sem).wait()
        pltpu.sync_copy(rows_v, out_hbm.at[pl.ds(base, CH)])
        return 0

    lax.fori_loop(0, NCH, body, 0)


BE = 2000


def _alpha_body(gsrc_ref, ea_ref, w1e_ref, a1_ref, al_ref):
    h = lax.dot_general(
        ea_ref[...], w1e_ref[...], (((1,), (1,)), ((), ())),
        preferred_element_type=jnp.float32)
    s = gsrc_ref[...] + h
    l = jnp.where(s >= 0, s, 0.01 * s)
    al_ref[...] = jnp.sum(l * a1_ref[...], axis=1, keepdims=True)


def _alpha(gsrc, ea, w1e, a1):
    return pl.pallas_call(
        _alpha_body,
        grid=(E // BE,),
        in_specs=[
            pl.BlockSpec((BE, D), lambda i: (i, 0)),
            pl.BlockSpec((BE, DE), lambda i: (i, 0)),
            pl.BlockSpec((D, DE), lambda i: (0, 0)),
            pl.BlockSpec((1, D), lambda i: (0, 0)),
        ],
        out_specs=pl.BlockSpec((BE, 1), lambda i: (i, 0)),
        out_shape=jax.ShapeDtypeStruct((E, 1), jnp.float32),
    )(gsrc, ea, w1e, a1)


def _shift_body(al_ref, out_ref):
    a = al_ref[...]
    out_ref[...] = a - jnp.max(a)


def _shift(al2d):
    return pl.pallas_call(
        _shift_body,
        out_shape=jax.ShapeDtypeStruct(al2d.shape, jnp.float32),
    )(al2d)


def _splat(v, j):
    dnums = lax.GatherDimensionNumbers(
        offset_dims=(), collapsed_slice_dims=(0,), start_index_map=(0,))
    idx = jnp.full((16, 1), j, jnp.int32)
    return lax.gather(v, idx, dnums, (1,),
                      mode=lax.GatherScatterMode.PROMISE_IN_BOUNDS)


@functools.cache
def _scatter_acc_kernel():
    return pl.kernel(
        _scatter_acc_body,
        out_type=jax.ShapeDtypeStruct((NC, NP, DP), jnp.float32),
        mesh=_mesh(),
        scratch_types=[
            pltpu.VMEM((CH,), jnp.int32),
            pltpu.VMEM((CH,), jnp.int32),
            pltpu.VMEM((CH,), jnp.float32),
            pltpu.VMEM((CH, DP), jnp.float32),
            pltpu.VMEM((ZR, DP), jnp.float32),
            pltpu.VMEM_SHARED((NP, DP), jnp.float32),
            pltpu.SemaphoreType.DMA,
        ],
        compiler_params=pltpu.CompilerParams(use_tc_tiling_on_sc=False),
    )


def _scatter_acc_body(xt_hbm, src_hbm, dst_hbm, al_hbm, zero_hbm, out_hbm,
                      srci, dsti, alv, rows, bounce, acc, sem):
    cid = lax.axis_index("c")
    sid = lax.axis_index("s")
    wid = sid * NC + cid
    base0 = wid * EPW
    nzch = NP // ZR

    pltpu.sync_copy(zero_hbm, bounce)

    def zbody(k, _):
        pltpu.sync_copy(bounce, acc.at[pl.ds((sid + k * NS) * ZR, ZR)])
        return 0

    lax.fori_loop(0, nzch // NS, zbody, 0)
    plsc.subcore_barrier()

    def body(i, _):
        base = base0 + i * CH
        pltpu.sync_copy(src_hbm.at[pl.ds(base, CH)], srci)
        pltpu.sync_copy(dst_hbm.at[pl.ds(base, CH)], dsti)
        pltpu.sync_copy(al_hbm.at[pl.ds(base, CH)], alv)
        pltpu.async_copy(xt_hbm.at[srci], rows, sem).wait()
        for b in range(CH // 16):
            w = jnp.exp(alv[pl.ds(b * 16, 16)])
            for j in range(16):
                ws = _splat(w, j)
                e = b * 16 + j
                for r in range(DP // 16):
                    rows[e, pl.ds(r * 16, 16)] = (
                        rows[e, pl.ds(r * 16, 16)] * ws)
        pltpu.sync_copy(rows, acc.at[dsti], add=True)
        return 0

    lax.fori_loop(0, NCH, body, 0)
    plsc.subcore_barrier()

    def dbody(k, _):
        r0 = (sid + k * NS) * ZR
        pltpu.sync_copy(acc.at[pl.ds(r0, ZR)], bounce)
        pltpu.sync_copy(bounce, out_hbm.at[cid, pl.ds(r0, ZR)])
        return 0

    lax.fori_loop(0, nzch // NS, dbody, 0)


def _final_body(p_ref, b_ref, o_ref):
    s = p_ref[0] + p_ref[1]
    num = s[:, :D]
    den = s[:, D:D + 1]
    o_ref[...] = jnp.where(den > 0, num / den, 0.0) + b_ref[...]


BF = 1024


def _final(parts, bias2d):
    return pl.pallas_call(
        _final_body,
        grid=(NP // BF,),
        in_specs=[
            pl.BlockSpec((NC, BF, DP), lambda i: (0, i, 0)),
            pl.BlockSpec((1, D), lambda i: (0, 0)),
        ],
        out_specs=pl.BlockSpec((BF, D), lambda i: (i, 0)),
        out_shape=jax.ShapeDtypeStruct((NP, D), jnp.float32),
    )(parts, bias2d)


def kernel(x, edge_index, edge_attr, W1, W2, attn, bias):
    src = edge_index[0]
    dst = edge_index[1]
    w1x = W1[:, :D]
    w1e = W1[:, D:]
    a1 = attn[:, :D]

    g, xt_ext = _prep(x, w1x, w2=W2)
    gsrc = _gather_rows_kernel()(g, src)
    al = _alpha(gsrc, edge_attr, w1e, a1)
    al_s = _shift(al.reshape(E // D, D)).reshape(E)
    zeros = jnp.zeros((ZR, DP), jnp.float32)
    parts = _scatter_acc_kernel()(xt_ext, src, dst, al_s, zeros)
    return _final(parts, bias.reshape(1, D))[:N]


# speedup vs baseline: 11.6678x; 1.4051x over previous
import functools

import jax
import jax.numpy as jnp
from jax import lax
from jax.experimental import pallas as pl
from jax.experimental.pallas import tpu as pltpu
from jax.experimental.pallas import tpu_sc as plsc

N = 10000
E = 320000
D = 128
DE = 16
DP = D + 16

NC = 2
NS = 16
NW = NC * NS
EPW = E // NW
CH = 80
NCH = EPW // CH
NP = 10240
ZR = 80

@functools.cache
def _mesh():
    return plsc.VectorSubcoreMesh(
        core_axis_name="c", subcore_axis_name="s", num_cores=NC, num_subcores=NS
    )


BN = 1000


def _prep_body(x_ref, w1x_ref, w2_ref, g_ref, xt_ref):
    xb = x_ref[...]
    g_ref[...] = lax.dot_general(
        xb, w1x_ref[...], (((1,), (1,)), ((), ())),
        preferred_element_type=jnp.float32)
    xt = lax.dot_general(
        xb, w2_ref[...], (((1,), (1,)), ((), ())),
        preferred_element_type=jnp.float32)
    pad = jnp.where(
        lax.broadcasted_iota(jnp.int32, (xb.shape[0], DP - D), 1) == 0,
        1.0, 0.0).astype(jnp.float32)
    xt_ref[...] = jnp.concatenate([xt, pad], axis=1)


def _prep(x, w1x, w2):
    return pl.pallas_call(
        _prep_body,
        grid=(N // BN,),
        in_specs=[
            pl.BlockSpec((BN, D), lambda i: (i, 0)),
            pl.BlockSpec((D, D), lambda i: (0, 0)),
            pl.BlockSpec((D, D), lambda i: (0, 0)),
        ],
        out_specs=[
            pl.BlockSpec((BN, D), lambda i: (i, 0)),
            pl.BlockSpec((BN, DP), lambda i: (i, 0)),
        ],
        out_shape=[
            jax.ShapeDtypeStruct((N, D), jnp.float32),
            jax.ShapeDtypeStruct((N, DP), jnp.float32),
        ],
    )(x, w1x, w2)


@functools.cache
def _gather_rows_kernel():
    return pl.kernel(
        _gather_rows_body,
        out_type=jax.ShapeDtypeStruct((E, D), jnp.float32),
        mesh=_mesh(),
        scratch_types=[
            pltpu.VMEM((NCH, CH), jnp.int32),
            pltpu.VMEM((CH, D), jnp.float32),
            pltpu.VMEM((CH, D), jnp.float32),
            pltpu.SemaphoreType.DMA,
            pltpu.SemaphoreType.DMA,
        ],
        compiler_params=pltpu.CompilerParams(use_tc_tiling_on_sc=False, needs_layout_passes=False),
    )


def _gather_rows_body(g_hbm, src3_hbm, out_hbm, idx_v, rows0, rows1, sg0, sg1):
    wid = lax.axis_index("s") * NC + lax.axis_index("c")
    base0 = wid * EPW

    pltpu.sync_copy(src3_hbm.at[wid], idx_v)

    pltpu.async_copy(g_hbm.at[idx_v.at[0]], rows0, sg0)

    def body(k, _):
        i0 = 2 * k
        i1 = i0 + 1

        @pl.when(i1 < NCH)
        def _():
            pltpu.async_copy(g_hbm.at[idx_v.at[i1]], rows1, sg1)

        pltpu.make_async_copy(g_hbm.at[idx_v.at[i0]], rows0, sg0).wait()
        pltpu.sync_copy(rows0, out_hbm.at[pl.ds(base0 + i0 * CH, CH)])

        @pl.when(i1 < NCH)
        def _():
            @pl.when(i1 + 1 < NCH)
            def _():
                pltpu.async_copy(g_hbm.at[idx_v.at[i1 + 1]], rows0, sg0)

            pltpu.make_async_copy(g_hbm.at[idx_v.at[i1]], rows1, sg1).wait()
            pltpu.sync_copy(rows1, out_hbm.at[pl.ds(base0 + i1 * CH, CH)])

        return 0

    lax.fori_loop(0, (NCH + 1) // 2, body, 0)


BE = 2000


def _alpha_body(gsrc_ref, ea_ref, w1e_ref, a1_ref, al_ref):
    h = lax.dot_general(
        ea_ref[...], w1e_ref[...], (((1,), (1,)), ((), ())),
        preferred_element_type=jnp.float32)
    s = gsrc_ref[...] + h
    l = jnp.where(s >= 0, s, 0.01 * s)
    al_ref[...] = jnp.sum(l * a1_ref[...], axis=1, keepdims=True)


def _alpha(gsrc, ea, w1e, a1):
    return pl.pallas_call(
        _alpha_body,
        grid=(E // BE,),
        in_specs=[
            pl.BlockSpec((BE, D), lambda i: (i, 0)),
            pl.BlockSpec((BE, DE), lambda i: (i, 0)),
            pl.BlockSpec((D, DE), lambda i: (0, 0)),
            pl.BlockSpec((1, D), lambda i: (0, 0)),
        ],
        out_specs=pl.BlockSpec((BE, 1), lambda i: (i, 0)),
        out_shape=jax.ShapeDtypeStruct((E, 1), jnp.float32),
    )(gsrc, ea, w1e, a1)


def _shift_body(al_ref, out_ref):
    a = al_ref[...]
    out_ref[...] = a - jnp.max(a)


def _shift(al2d):
    return pl.pallas_call(
        _shift_body,
        out_shape=jax.ShapeDtypeStruct(al2d.shape, jnp.float32),
    )(al2d)


def _splat(v, j):
    dnums = lax.GatherDimensionNumbers(
        offset_dims=(), collapsed_slice_dims=(0,), start_index_map=(0,))
    idx = jnp.full((16, 1), j, jnp.int32)
    return lax.gather(v, idx, dnums, (1,),
                      mode=lax.GatherScatterMode.PROMISE_IN_BOUNDS)


@functools.cache
def _scatter_acc_kernel():
    return pl.kernel(
        _scatter_acc_body,
        out_type=jax.ShapeDtypeStruct((NC, NP, DP), jnp.float32),
        mesh=_mesh(),
        scratch_types=[
            pltpu.VMEM((NCH, CH), jnp.int32),
            pltpu.VMEM((2, CH), jnp.int32),
            pltpu.VMEM((2, CH), jnp.int32),
            pltpu.VMEM((CH, DP), jnp.float32),
            pltpu.VMEM((CH, DP), jnp.float32),
            pltpu.VMEM_SHARED((NP, DP), jnp.float32),
            pltpu.SemaphoreType.DMA,
            pltpu.SemaphoreType.DMA,
            pltpu.SemaphoreType.DMA,
            pltpu.SemaphoreType.DMA,
            pltpu.SemaphoreType.DMA,
            pltpu.SemaphoreType.DMA,
        ],
        compiler_params=pltpu.CompilerParams(use_tc_tiling_on_sc=False, needs_layout_passes=False),
    )


def _scatter_acc_body(xt_hbm, src3_hbm, dal_hbm, zero_hbm, out_hbm,
                      srci, da0, da1, rows0, rows1, acc,
                      sd0, sd1, sg0, sg1, ss0, ss1):
    cid = lax.axis_index("c")
    sid = lax.axis_index("s")
    wid = sid * NC + cid
    nzch = NP // ZR

    pltpu.sync_copy(src3_hbm.at[wid], srci)

    pltpu.sync_copy(zero_hbm, rows0)

    def zbody(k, _):
        pltpu.sync_copy(rows0, acc.at[pl.ds((sid + k * NS) * ZR, ZR)])
        return 0

    lax.fori_loop(0, nzch // NS, zbody, 0)
    plsc.subcore_barrier()

    def compute(i, rows, da):
        for b in range(CH // 16):
            w = jnp.exp(plsc.bitcast(da[1, pl.ds(b * 16, 16)], jnp.float32))
            for j in range(16):
                ws = _splat(w, j)
                e = b * 16 + j
                for r in range(DP // 16):
                    rows[e, pl.ds(r * 16, 16)] = (
                        rows[e, pl.ds(r * 16, 16)] * ws)

    pltpu.async_copy(dal_hbm.at[wid, 0], da0, sd0)
    pltpu.async_copy(xt_hbm.at[srci.at[0]], rows0, sg0)

    def slot(i, da, rows, sd, sg, ss, da_n, rows_n, sd_n, sg_n, ss_n):
        @pl.when(i + 1 < NCH)
        def _():
            @pl.when(i >= 1)
            def _():
                pltpu.make_async_copy(rows_n, acc.at[da_n.at[0]], ss_n).wait()

            pltpu.async_copy(dal_hbm.at[wid, i + 1], da_n, sd_n)
            pltpu.async_copy(xt_hbm.at[srci.at[i + 1]], rows_n, sg_n)

        pltpu.make_async_copy(xt_hbm.at[srci.at[i]], rows, sg).wait()
        pltpu.make_async_copy(dal_hbm.at[wid, i], da, sd).wait()
        compute(i, rows, da)
        pltpu.async_copy(rows, acc.at[da.at[0]], ss, add=True)

    def body(k, _):
        i0 = 2 * k
        i1 = i0 + 1
        slot(i0, da0, rows0, sd0, sg0, ss0, da1, rows1, sd1, sg1, ss1)

        @pl.when(i1 < NCH)
        def _():
            slot(i1, da1, rows1, sd1, sg1, ss1, da0, rows0, sd0, sg0, ss0)

        return 0

    lax.fori_loop(0, (NCH + 1) // 2, body, 0)
    pltpu.make_async_copy(rows0, acc.at[da0.at[0]], ss0).wait()
    pltpu.make_async_copy(rows1, acc.at[da1.at[0]], ss1).wait()
    plsc.subcore_barrier()

    def dbody(k, _):
        r0 = (sid + k * NS) * ZR
        pltpu.sync_copy(acc.at[pl.ds(r0, ZR)], rows0)
        pltpu.sync_copy(rows0, out_hbm.at[cid, pl.ds(r0, ZR)])
        return 0

    lax.fori_loop(0, nzch // NS, dbody, 0)


def _final_body(p_ref, b_ref, o_ref):
    s = p_ref[0] + p_ref[1]
    num = s[:, :D]
    den = s[:, D:D + 1]
    o_ref[...] = jnp.where(den > 0, num / den, 0.0) + b_ref[...]


BF = 1024


def _final(parts, bias2d):
    return pl.pallas_call(
        _final_body,
        grid=(NP // BF,),
        in_specs=[
            pl.BlockSpec((NC, BF, DP), lambda i: (0, i, 0)),
            pl.BlockSpec((1, D), lambda i: (0, 0)),
        ],
        out_specs=pl.BlockSpec((BF, D), lambda i: (i, 0)),
        out_shape=jax.ShapeDtypeStruct((NP, D), jnp.float32),
    )(parts, bias2d)


def kernel(x, edge_index, edge_attr, W1, W2, attn, bias):
    src = edge_index[0]
    dst = edge_index[1]
    src3 = src.reshape(NW, NCH, CH)
    dst3 = dst.reshape(NW, NCH, CH)
    w1x = W1[:, :D]
    w1e = W1[:, D:]
    a1 = attn[:, :D]

    g, xt_ext = _prep(x, w1x, w2=W2)
    gsrc = _gather_rows_kernel()(g, src3)
    al = _alpha(gsrc, edge_attr, w1e, a1)
    al_s = _shift(al.reshape(E // D, D)).reshape(NW, NCH, CH)
    dal = jnp.stack(
        [dst3, lax.bitcast_convert_type(al_s, jnp.int32)], axis=2)
    zeros = jnp.zeros((ZR, DP), jnp.float32)
    parts = _scatter_acc_kernel()(xt_ext, src3, dal, zeros)
    return _final(parts, bias.reshape(1, D))[:N]


# speedup vs baseline: 12.1705x; 1.0431x over previous
import functools

import jax
import jax.numpy as jnp
from jax import lax
from jax.experimental import pallas as pl
from jax.experimental.pallas import tpu as pltpu
from jax.experimental.pallas import tpu_sc as plsc

N = 10000
E = 320000
D = 128
DE = 16
DP = D + 16

NC = 2
NS = 16
NW = NC * NS
EPW = E // NW
CH = 80
NCH = EPW // CH
NP = 10240
ZR = 80

@functools.cache
def _mesh():
    return plsc.VectorSubcoreMesh(
        core_axis_name="c", subcore_axis_name="s", num_cores=NC, num_subcores=NS
    )


BN = 1000


def _prep_body(x_ref, w1x_ref, w2_ref, g_ref, xt_ref):
    xb = x_ref[...]
    g_ref[...] = lax.dot_general(
        xb, w1x_ref[...], (((1,), (1,)), ((), ())),
        preferred_element_type=jnp.float32)
    xt = lax.dot_general(
        xb, w2_ref[...], (((1,), (1,)), ((), ())),
        preferred_element_type=jnp.float32)
    pad = jnp.where(
        lax.broadcasted_iota(jnp.int32, (xb.shape[0], DP - D), 1) == 0,
        1.0, 0.0).astype(jnp.float32)
    xt_ref[...] = jnp.concatenate([xt, pad], axis=1)


def _prep(x, w1x, w2):
    return pl.pallas_call(
        _prep_body,
        grid=(N // BN,),
        in_specs=[
            pl.BlockSpec((BN, D), lambda i: (i, 0)),
            pl.BlockSpec((D, D), lambda i: (0, 0)),
            pl.BlockSpec((D, D), lambda i: (0, 0)),
        ],
        out_specs=[
            pl.BlockSpec((BN, D), lambda i: (i, 0)),
            pl.BlockSpec((BN, DP), lambda i: (i, 0)),
        ],
        out_shape=[
            jax.ShapeDtypeStruct((N, D), jnp.float32),
            jax.ShapeDtypeStruct((N, DP), jnp.float32),
        ],
    )(x, w1x, w2)


@functools.cache
def _gather_rows_kernel():
    return pl.kernel(
        _gather_rows_body,
        out_type=jax.ShapeDtypeStruct((E, D), jnp.float32),
        mesh=_mesh(),
        scratch_types=[
            pltpu.VMEM((NCH, CH), jnp.int32),
            pltpu.VMEM((CH, D), jnp.float32),
            pltpu.VMEM((CH, D), jnp.float32),
            pltpu.SemaphoreType.DMA,
            pltpu.SemaphoreType.DMA,
        ],
        compiler_params=pltpu.CompilerParams(use_tc_tiling_on_sc=False, needs_layout_passes=False),
    )


def _gather_rows_body(g_hbm, src3_hbm, out_hbm, idx_v, rows0, rows1, sg0, sg1):
    wid = lax.axis_index("s") * NC + lax.axis_index("c")
    base0 = wid * EPW

    pltpu.sync_copy(src3_hbm.at[wid], idx_v)

    pltpu.async_copy(g_hbm.at[idx_v.at[0]], rows0, sg0)

    def body(k, _):
        i0 = 2 * k
        i1 = i0 + 1

        @pl.when(i1 < NCH)
        def _():
            pltpu.async_copy(g_hbm.at[idx_v.at[i1]], rows1, sg1)

        pltpu.make_async_copy(g_hbm.at[idx_v.at[i0]], rows0, sg0).wait()
        pltpu.sync_copy(rows0, out_hbm.at[pl.ds(base0 + i0 * CH, CH)])

        @pl.when(i1 < NCH)
        def _():
            @pl.when(i1 + 1 < NCH)
            def _():
                pltpu.async_copy(g_hbm.at[idx_v.at[i1 + 1]], rows0, sg0)

            pltpu.make_async_copy(g_hbm.at[idx_v.at[i1]], rows1, sg1).wait()
            pltpu.sync_copy(rows1, out_hbm.at[pl.ds(base0 + i1 * CH, CH)])

        return 0

    lax.fori_loop(0, (NCH + 1) // 2, body, 0)


BE = 2000


def _alpha_body(gsrc_ref, ea_ref, w1e_ref, a1_ref, al_ref):
    h = lax.dot_general(
        ea_ref[...], w1e_ref[...], (((1,), (1,)), ((), ())),
        preferred_element_type=jnp.float32)
    s = gsrc_ref[...] + h
    l = jnp.where(s >= 0, s, 0.01 * s)
    al = lax.dot_general(
        l, a1_ref[...], (((1,), (1,)), ((), ())),
        preferred_element_type=jnp.float32)
    al_ref[...] = jnp.minimum(al, 80.0)


def _alpha(gsrc, ea, w1e, a1):
    return pl.pallas_call(
        _alpha_body,
        grid=(E // BE,),
        in_specs=[
            pl.BlockSpec((BE, D), lambda i: (i, 0)),
            pl.BlockSpec((BE, DE), lambda i: (i, 0)),
            pl.BlockSpec((D, DE), lambda i: (0, 0)),
            pl.BlockSpec((1, D), lambda i: (0, 0)),
        ],
        out_specs=pl.BlockSpec((BE, 1), lambda i: (i, 0)),
        out_shape=jax.ShapeDtypeStruct((E, 1), jnp.float32),
    )(gsrc, ea, w1e, a1)


def _splat(v, j):
    dnums = lax.GatherDimensionNumbers(
        offset_dims=(), collapsed_slice_dims=(0,), start_index_map=(0,))
    idx = jnp.full((16, 1), j, jnp.int32)
    return lax.gather(v, idx, dnums, (1,),
                      mode=lax.GatherScatterMode.PROMISE_IN_BOUNDS)


@functools.cache
def _scatter_acc_kernel():
    return pl.kernel(
        _scatter_acc_body,
        out_type=jax.ShapeDtypeStruct((NC, NP, DP), jnp.float32),
        mesh=_mesh(),
        scratch_types=[
            pltpu.VMEM((NCH, CH), jnp.int32),
            pltpu.VMEM((CH,), jnp.int32),
            pltpu.VMEM((CH,), jnp.int32),
            pltpu.VMEM((CH,), jnp.float32),
            pltpu.VMEM((CH,), jnp.float32),
            pltpu.VMEM((CH, DP), jnp.float32),
            pltpu.VMEM((CH, DP), jnp.float32),
            pltpu.VMEM_SHARED((NP, DP), jnp.float32),
            pltpu.SemaphoreType.DMA,
            pltpu.SemaphoreType.DMA,
            pltpu.SemaphoreType.DMA,
            pltpu.SemaphoreType.DMA,
            pltpu.SemaphoreType.DMA,
            pltpu.SemaphoreType.DMA,
        ],
        compiler_params=pltpu.CompilerParams(use_tc_tiling_on_sc=False, needs_layout_passes=False),
    )


def _scatter_acc_body(xt_hbm, src3_hbm, dst3_hbm, al3_hbm, zero_hbm, out_hbm,
                      srci, db0, db1, ab0, ab1, rows0, rows1, acc,
                      sd0, sd1, sg0, sg1, ss0, ss1):
    cid = lax.axis_index("c")
    sid = lax.axis_index("s")
    wid = sid * NC + cid
    nzch = NP // ZR

    pltpu.sync_copy(src3_hbm.at[wid], srci)

    pltpu.sync_copy(zero_hbm, rows0)

    def zbody(k, _):
        pltpu.sync_copy(rows0, acc.at[pl.ds((sid + k * NS) * ZR, ZR)])
        return 0

    lax.fori_loop(0, nzch // NS, zbody, 0)
    plsc.subcore_barrier()

    def compute(rows, ab):
        for b in range(CH // 16):
            w = jnp.exp(ab[pl.ds(b * 16, 16)])
            for j in range(16):
                ws = _splat(w, j)
                e = b * 16 + j
                for r in range(DP // 16):
                    rows[e, pl.ds(r * 16, 16)] = (
                        rows[e, pl.ds(r * 16, 16)] * ws)

    pltpu.async_copy(dst3_hbm.at[wid, 0], db0, sd0)
    pltpu.async_copy(al3_hbm.at[wid, 0], ab0, sd0)
    pltpu.async_copy(xt_hbm.at[srci.at[0]], rows0, sg0)

    def slot(i, db, ab, rows, sd, sg, ss, db_n, ab_n, rows_n,
             sd_n, sg_n, ss_n):
        @pl.when(i + 1 < NCH)
        def _():
            @pl.when(i >= 1)
            def _():
                pltpu.make_async_copy(rows_n, acc.at[db_n], ss_n).wait()

            pltpu.async_copy(dst3_hbm.at[wid, i + 1], db_n, sd_n)
            pltpu.async_copy(al3_hbm.at[wid, i + 1], ab_n, sd_n)
            pltpu.async_copy(xt_hbm.at[srci.at[i + 1]], rows_n, sg_n)

        pltpu.make_async_copy(xt_hbm.at[srci.at[i]], rows, sg).wait()
        pltpu.make_async_copy(dst3_hbm.at[wid, i], db, sd).wait()
        pltpu.make_async_copy(al3_hbm.at[wid, i], ab, sd).wait()
        compute(rows, ab)
        pltpu.async_copy(rows, acc.at[db], ss, add=True)

    def body(k, _):
        i0 = 2 * k
        i1 = i0 + 1
        slot(i0, db0, ab0, rows0, sd0, sg0, ss0, db1, ab1, rows1,
             sd1, sg1, ss1)

        @pl.when(i1 < NCH)
        def _():
            slot(i1, db1, ab1, rows1, sd1, sg1, ss1, db0, ab0, rows0,
                 sd0, sg0, ss0)

        return 0

    lax.fori_loop(0, (NCH + 1) // 2, body, 0)
    pltpu.make_async_copy(rows0, acc.at[db0], ss0).wait()
    pltpu.make_async_copy(rows1, acc.at[db1], ss1).wait()
    plsc.subcore_barrier()

    def dbody(k, _):
        r0 = (sid + k * NS) * ZR
        pltpu.sync_copy(acc.at[pl.ds(r0, ZR)], rows0)
        pltpu.sync_copy(rows0, out_hbm.at[cid, pl.ds(r0, ZR)])
        return 0

    lax.fori_loop(0, nzch // NS, dbody, 0)


def _final_body(p_ref, b_ref, o_ref):
    s = p_ref[0] + p_ref[1]
    num = s[:, :D]
    den = s[:, D:D + 1]
    o_ref[...] = jnp.where(den > 0, num / den, 0.0) + b_ref[...]


BF = 1024


def _final(parts, bias2d):
    return pl.pallas_call(
        _final_body,
        grid=(NP // BF,),
        in_specs=[
            pl.BlockSpec((NC, BF, DP), lambda i: (0, i, 0)),
            pl.BlockSpec((1, D), lambda i: (0, 0)),
        ],
        out_specs=pl.BlockSpec((BF, D), lambda i: (i, 0)),
        out_shape=jax.ShapeDtypeStruct((NP, D), jnp.float32),
    )(parts, bias2d)


def kernel(x, edge_index, edge_attr, W1, W2, attn, bias):
    src = edge_index[0]
    dst = edge_index[1]
    src3 = src.reshape(NW, NCH, CH)
    dst3 = dst.reshape(NW, NCH, CH)
    w1x = W1[:, :D]
    w1e = W1[:, D:]
    a1 = attn[:, :D]

    g, xt_ext = _prep(x, w1x, w2=W2)
    gsrc = _gather_rows_kernel()(g, src3)
    al3 = _alpha(gsrc, edge_attr, w1e, a1).reshape(NW, NCH, CH)
    zeros = jnp.zeros((ZR, DP), jnp.float32)
    parts = _scatter_acc_kernel()(xt_ext, src3, dst3, al3, zeros)
    return _final(parts, bias.reshape(1, D))[:N]


# speedup vs baseline: 15.5112x; 1.2745x over previous
import functools

import jax
import jax.numpy as jnp
from jax import lax
from jax.experimental import pallas as pl
from jax.experimental.pallas import tpu as pltpu
from jax.experimental.pallas import tpu_sc as plsc

N = 10000
E = 320000
D = 128
DE = 16
DP = D + 16

NC = 2
NS = 16
NW = NC * NS
EPW = E // NW
CH = 80
NCH = EPW // CH
NP = 10240
ZR = 80

@functools.cache
def _mesh():
    return plsc.VectorSubcoreMesh(
        core_axis_name="c", subcore_axis_name="s", num_cores=NC, num_subcores=NS
    )


BN = 1000


def _prep_body(x_ref, w1x_ref, w2_ref, g_ref, xt_ref):
    xb = x_ref[...]
    g_ref[...] = lax.dot_general(
        xb, w1x_ref[...], (((1,), (1,)), ((), ())),
        preferred_element_type=jnp.float32)
    xt = lax.dot_general(
        xb, w2_ref[...], (((1,), (1,)), ((), ())),
        preferred_element_type=jnp.float32)
    pad = jnp.where(
        lax.broadcasted_iota(jnp.int32, (xb.shape[0], DP - D), 1) == 0,
        1.0, 0.0).astype(jnp.float32)
    xt_ref[...] = jnp.concatenate([xt, pad], axis=1)


def _prep(x, w1x, w2):
    return pl.pallas_call(
        _prep_body,
        grid=(N // BN,),
        in_specs=[
            pl.BlockSpec((BN, D), lambda i: (i, 0)),
            pl.BlockSpec((D, D), lambda i: (0, 0)),
            pl.BlockSpec((D, D), lambda i: (0, 0)),
        ],
        out_specs=[
            pl.BlockSpec((BN, D), lambda i: (i, 0)),
            pl.BlockSpec((BN, DP), lambda i: (i, 0)),
        ],
        out_shape=[
            jax.ShapeDtypeStruct((N, D), jnp.float32),
            jax.ShapeDtypeStruct((N, DP), jnp.float32),
        ],
    )(x, w1x, w2)


@functools.cache
def _gather_rows_kernel():
    return pl.kernel(
        _gather_rows_body,
        out_type=jax.ShapeDtypeStruct((E, D), jnp.float32),
        mesh=_mesh(),
        scratch_types=[
            pltpu.VMEM((NCH, CH), jnp.int32),
            pltpu.VMEM((CH, D), jnp.float32),
            pltpu.VMEM((CH, D), jnp.float32),
            pltpu.SemaphoreType.DMA,
            pltpu.SemaphoreType.DMA,
        ],
        compiler_params=pltpu.CompilerParams(use_tc_tiling_on_sc=False, needs_layout_passes=False),
    )


def _gather_rows_body(g_hbm, src3_hbm, out_hbm, idx_v, rows0, rows1, sg0, sg1):
    wid = lax.axis_index("s") * NC + lax.axis_index("c")
    base0 = wid * EPW

    pltpu.sync_copy(src3_hbm.at[wid], idx_v)

    pltpu.async_copy(g_hbm.at[idx_v.at[0]], rows0, sg0)

    def body(k, _):
        i0 = 2 * k
        i1 = i0 + 1

        @pl.when(i1 < NCH)
        def _():
            pltpu.async_copy(g_hbm.at[idx_v.at[i1]], rows1, sg1)

        pltpu.make_async_copy(g_hbm.at[idx_v.at[i0]], rows0, sg0).wait()
        pltpu.sync_copy(rows0, out_hbm.at[pl.ds(base0 + i0 * CH, CH)])

        @pl.when(i1 < NCH)
        def _():
            @pl.when(i1 + 1 < NCH)
            def _():
                pltpu.async_copy(g_hbm.at[idx_v.at[i1 + 1]], rows0, sg0)

            pltpu.make_async_copy(g_hbm.at[idx_v.at[i1]], rows1, sg1).wait()
            pltpu.sync_copy(rows1, out_hbm.at[pl.ds(base0 + i1 * CH, CH)])

        return 0

    lax.fori_loop(0, (NCH + 1) // 2, body, 0)


BE = 8192
NBA = 40


def _alpha_body(gsrc_ref, ea_ref, w1e_ref, a1_ref, al_ref):
    h = lax.dot_general(
        ea_ref[...], w1e_ref[...], (((1,), (1,)), ((), ())),
        preferred_element_type=jnp.float32)
    s = gsrc_ref[...] + h
    l = jnp.where(s >= 0, s, 0.01 * s)
    al = lax.dot_general(
        l, a1_ref[...], (((1,), (1,)), ((), ())),
        preferred_element_type=jnp.float32)
    al_ref[...] = jnp.reshape(jnp.minimum(al, 80.0), (BE // D, D))


def _alpha(gsrc, ea, w1e, a1):
    return pl.pallas_call(
        _alpha_body,
        grid=(NBA,),
        in_specs=[
            pl.BlockSpec((BE, D), lambda i: (i, 0)),
            pl.BlockSpec((BE, DE), lambda i: (i, 0)),
            pl.BlockSpec((D, DE), lambda i: (0, 0)),
            pl.BlockSpec((1, D), lambda i: (0, 0)),
        ],
        out_specs=pl.BlockSpec((BE // D, D), lambda i: (i, 0)),
        out_shape=jax.ShapeDtypeStruct((NBA * BE // D, D), jnp.float32),
    )(gsrc, ea, w1e, a1)


def _splat(v, j):
    dnums = lax.GatherDimensionNumbers(
        offset_dims=(), collapsed_slice_dims=(0,), start_index_map=(0,))
    idx = jnp.full((16, 1), j, jnp.int32)
    return lax.gather(v, idx, dnums, (1,),
                      mode=lax.GatherScatterMode.PROMISE_IN_BOUNDS)


@functools.cache
def _scatter_acc_kernel():
    return pl.kernel(
        _scatter_acc_body,
        out_type=jax.ShapeDtypeStruct((NC, NP, DP), jnp.float32),
        mesh=_mesh(),
        scratch_types=[
            pltpu.VMEM((NCH, CH), jnp.int32),
            pltpu.VMEM((CH,), jnp.int32),
            pltpu.VMEM((CH,), jnp.int32),
            pltpu.VMEM((CH,), jnp.float32),
            pltpu.VMEM((CH,), jnp.float32),
            pltpu.VMEM((CH, DP), jnp.float32),
            pltpu.VMEM((CH, DP), jnp.float32),
            pltpu.VMEM_SHARED((NP, DP), jnp.float32),
            pltpu.SemaphoreType.DMA,
            pltpu.SemaphoreType.DMA,
            pltpu.SemaphoreType.DMA,
            pltpu.SemaphoreType.DMA,
            pltpu.SemaphoreType.DMA,
            pltpu.SemaphoreType.DMA,
        ],
        compiler_params=pltpu.CompilerParams(use_tc_tiling_on_sc=False, needs_layout_passes=False),
    )


def _scatter_acc_body(xt_hbm, src3_hbm, dst3_hbm, al3_hbm, zero_hbm, out_hbm,
                      srci, db0, db1, ab0, ab1, rows0, rows1, acc,
                      sd0, sd1, sg0, sg1, ss0, ss1):
    cid = lax.axis_index("c")
    sid = lax.axis_index("s")
    wid = sid * NC + cid
    nzch = NP // ZR

    pltpu.sync_copy(src3_hbm.at[wid], srci)

    pltpu.sync_copy(zero_hbm, rows0)

    def zbody(k, _):
        pltpu.sync_copy(rows0, acc.at[pl.ds((sid + k * NS) * ZR, ZR)])
        return 0

    lax.fori_loop(0, nzch // NS, zbody, 0)
    plsc.subcore_barrier()

    def compute(rows, ab):
        for b in range(CH // 16):
            w = jnp.exp(ab[pl.ds(b * 16, 16)])
            for j in range(16):
                ws = _splat(w, j)
                e = b * 16 + j
                for r in range(DP // 16):
                    rows[e, pl.ds(r * 16, 16)] = (
                        rows[e, pl.ds(r * 16, 16)] * ws)

    pltpu.async_copy(dst3_hbm.at[wid, 0], db0, sd0)
    pltpu.async_copy(al3_hbm.at[wid, 0], ab0, sd0)
    pltpu.async_copy(xt_hbm.at[srci.at[0]], rows0, sg0)

    def slot(i, db, ab, rows, sd, sg, ss, db_n, ab_n, rows_n,
             sd_n, sg_n, ss_n):
        @pl.when(i + 1 < NCH)
        def _():
            @pl.when(i >= 1)
            def _():
                pltpu.make_async_copy(rows_n, acc.at[db_n], ss_n).wait()

            pltpu.async_copy(dst3_hbm.at[wid, i + 1], db_n, sd_n)
            pltpu.async_copy(al3_hbm.at[wid, i + 1], ab_n, sd_n)
            pltpu.async_copy(xt_hbm.at[srci.at[i + 1]], rows_n, sg_n)

        pltpu.make_async_copy(xt_hbm.at[srci.at[i]], rows, sg).wait()
        pltpu.make_async_copy(dst3_hbm.at[wid, i], db, sd).wait()
        pltpu.make_async_copy(al3_hbm.at[wid, i], ab, sd).wait()
        compute(rows, ab)
        pltpu.async_copy(rows, acc.at[db], ss, add=True)

    def body(k, _):
        i0 = 2 * k
        i1 = i0 + 1
        slot(i0, db0, ab0, rows0, sd0, sg0, ss0, db1, ab1, rows1,
             sd1, sg1, ss1)

        @pl.when(i1 < NCH)
        def _():
            slot(i1, db1, ab1, rows1, sd1, sg1, ss1, db0, ab0, rows0,
                 sd0, sg0, ss0)

        return 0

    lax.fori_loop(0, (NCH + 1) // 2, body, 0)
    pltpu.make_async_copy(rows0, acc.at[db0], ss0).wait()
    pltpu.make_async_copy(rows1, acc.at[db1], ss1).wait()
    plsc.subcore_barrier()

    def dbody(k, _):
        r0 = (sid + k * NS) * ZR
        pltpu.sync_copy(acc.at[pl.ds(r0, ZR)], rows0)
        pltpu.sync_copy(rows0, out_hbm.at[cid, pl.ds(r0, ZR)])
        return 0

    lax.fori_loop(0, nzch // NS, dbody, 0)


def _final_body(p_ref, b_ref, o_ref):
    s = p_ref[0] + p_ref[1]
    num = s[:, :D]
    den = s[:, D:D + 1]
    o_ref[...] = jnp.where(den > 0, num / den, 0.0) + b_ref[...]


BF = 1024


def _final(parts, bias2d):
    return pl.pallas_call(
        _final_body,
        grid=(NP // BF,),
        in_specs=[
            pl.BlockSpec((NC, BF, DP), lambda i: (0, i, 0)),
            pl.BlockSpec((1, D), lambda i: (0, 0)),
        ],
        out_specs=pl.BlockSpec((BF, D), lambda i: (i, 0)),
        out_shape=jax.ShapeDtypeStruct((NP, D), jnp.float32),
    )(parts, bias2d)


def kernel(x, edge_index, edge_attr, W1, W2, attn, bias):
    src = edge_index[0]
    dst = edge_index[1]
    src3 = src.reshape(NW, NCH, CH)
    dst3 = dst.reshape(NW, NCH, CH)
    w1x = W1[:, :D]
    w1e = W1[:, D:]
    a1 = attn[:, :D]

    g, xt_ext = _prep(x, w1x, w2=W2)
    gsrc = _gather_rows_kernel()(g, src3)
    al3 = _alpha(gsrc, edge_attr, w1e, a1)[:E // D].reshape(NW, NCH, CH)
    zeros = jnp.zeros((ZR, DP), jnp.float32)
    parts = _scatter_acc_kernel()(xt_ext, src3, dst3, al3, zeros)
    return _final(parts, bias.reshape(1, D))[:N]
